# baseline (speedup 1.0000x reference)
import numpy as np

import jax
import jax.numpy as jnp
from jax.experimental import pallas as pl
from jax.experimental.pallas import tpu as pltpu

_ROT_A = (13, 15, 26, 6)
_ROT_B = (17, 29, 16, 24)
_TINY = np.float32(np.finfo(np.float32).tiny)
_ONE = np.float32(1.0)
_NEG_INF = np.float32(-np.inf)


def _tf_rounds(x0, x1, rots):
    for r in rots:
        x0 = x0 + x1
        x1 = (x1 << jnp.uint32(r)) | (x1 >> jnp.uint32(32 - r))
        x1 = x0 ^ x1
    return x0, x1


def _threefry_bits(k1, k2, idx):
    ks2 = k1 ^ k2 ^ jnp.uint32(0x1BD11BDA)
    x0 = jnp.zeros_like(idx) + k1
    x1 = idx + k2
    x0, x1 = _tf_rounds(x0, x1, _ROT_A)
    x0 = x0 + k2
    x1 = x1 + ks2 + jnp.uint32(1)
    x0, x1 = _tf_rounds(x0, x1, _ROT_B)
    x0 = x0 + ks2
    x1 = x1 + k1 + jnp.uint32(2)
    x0, x1 = _tf_rounds(x0, x1, _ROT_A)
    x0 = x0 + k1
    x1 = x1 + k2 + jnp.uint32(3)
    x0, x1 = _tf_rounds(x0, x1, _ROT_B)
    x0 = x0 + k2
    x1 = x1 + ks2 + jnp.uint32(4)
    x0, x1 = _tf_rounds(x0, x1, _ROT_A)
    x0 = x0 + ks2
    x1 = x1 + k1 + jnp.uint32(5)
    return x0 ^ x1


def _sampler_body(vocab, block, nblk, kd_ref, t_ref, logits_ref, tok_ref,
                  lp_ref, gidx, ymax, yidx, ysel, lm, ls):
    j = pl.program_id(0)
    rows = logits_ref.shape[0]

    x = logits_ref[...]
    t = t_ref[...]
    safe_t = jnp.where(t == 0.0, _ONE, t)
    scaled = x / safe_t

    col = jax.lax.broadcasted_iota(jnp.int32, (rows, block), 1) + j * block
    row = jax.lax.broadcasted_iota(jnp.int32, (rows, block), 0)
    valid = col < vocab

    k1 = kd_ref[0]
    k2 = kd_ref[1]
    idx = (row * vocab + col).astype(jnp.uint32)
    bits = _threefry_bits(k1, k2, idx)
    fbits = (bits >> jnp.uint32(9)) | jnp.uint32(0x3F800000)
    u01 = jax.lax.bitcast_convert_type(fbits, jnp.float32) - _ONE
    u = jnp.maximum(_TINY, u01 * (_ONE - _TINY) + _TINY)
    g = -jnp.log(-jnp.log(u))

    sm = jnp.where(valid, scaled, _NEG_INF)
    ym = sm + g

    big = jnp.int32(0x7FFFFFFF)
    bm = jnp.max(sm, axis=1, keepdims=True)
    bxi = jnp.min(jnp.where(sm == bm, col, big), axis=1, keepdims=True)
    by = jnp.max(ym, axis=1, keepdims=True)
    byi = jnp.min(jnp.where(ym == by, col, big), axis=1, keepdims=True)
    bysel = jnp.max(jnp.where(col == byi, sm, _NEG_INF), axis=1, keepdims=True)
    bs = jnp.sum(jnp.exp(sm - bm), axis=1, keepdims=True)

    @pl.when(j == 0)
    def _init():
        gidx[...] = bxi
        ymax[...] = by
        yidx[...] = byi
        ysel[...] = bysel
        lm[...] = bm
        ls[...] = bs

    @pl.when(j > 0)
    def _merge():
        m_old = lm[...]
        upg = bm > m_old
        gidx[...] = jnp.where(upg, bxi, gidx[...])
        upy = by > ymax[...]
        yidx[...] = jnp.where(upy, byi, yidx[...])
        ysel[...] = jnp.where(upy, bysel, ysel[...])
        ymax[...] = jnp.where(upy, by, ymax[...])
        m_new = jnp.where(upg, bm, m_old)
        ls[...] = ls[...] * jnp.exp(m_old - m_new) + bs * jnp.exp(bm - m_new)
        lm[...] = m_new

    @pl.when(j == nblk - 1)
    def _finish():
        zero_t = t == 0.0
        tok_ref[...] = jnp.where(zero_t, gidx[...], yidx[...])
        sel = jnp.where(zero_t, lm[...], ysel[...])
        log_z = lm[...] + jnp.log(ls[...])
        lp_ref[...] = sel - log_z


def kernel(logits, temperatures, key):
    rows, vocab = logits.shape
    logits = logits.astype(jnp.float32)
    kd = jax.random.key_data(key).astype(jnp.uint32).reshape(2)
    t2 = temperatures.astype(jnp.float32).reshape(rows, 1)

    block = 4096 if vocab > 4096 else max(128, -(-vocab // 128) * 128)
    nblk = -(-vocab // block)

    fn = lambda *a: _sampler_body(vocab, block, nblk, *a)
    tok, lp = pl.pallas_call(
        fn,
        grid=(nblk,),
        in_specs=[
            pl.BlockSpec(memory_space=pltpu.SMEM),
            pl.BlockSpec((rows, 1), lambda j: (0, 0)),
            pl.BlockSpec((rows, block), lambda j: (0, j)),
        ],
        out_specs=[
            pl.BlockSpec((rows, 1), lambda j: (0, 0)),
            pl.BlockSpec((rows, 1), lambda j: (0, 0)),
        ],
        out_shape=[
            jax.ShapeDtypeStruct((rows, 1), jnp.int32),
            jax.ShapeDtypeStruct((rows, 1), jnp.float32),
        ],
        scratch_shapes=[
            pltpu.VMEM((rows, 1), jnp.int32),
            pltpu.VMEM((rows, 1), jnp.float32),
            pltpu.VMEM((rows, 1), jnp.int32),
            pltpu.VMEM((rows, 1), jnp.float32),
            pltpu.VMEM((rows, 1), jnp.float32),
            pltpu.VMEM((rows, 1), jnp.float32),
        ],
        compiler_params=pltpu.CompilerParams(
            dimension_semantics=("arbitrary",),
        ),
    )(kd, t2, logits)
    return tok.reshape(rows), lp.reshape(rows)


# speedup vs baseline: 1.1337x; 1.1337x over previous
import numpy as np

import jax
import jax.numpy as jnp
from jax.experimental import pallas as pl
from jax.experimental.pallas import tpu as pltpu

_ROT_A = (13, 15, 26, 6)
_ROT_B = (17, 29, 16, 24)
_TINY = np.float32(np.finfo(np.float32).tiny)
_ONE = np.float32(1.0)
_NEG_INF = np.float32(-np.inf)


def _tf_rounds(x0, x1, rots):
    for r in rots:
        x0 = x0 + x1
        x1 = (x1 << jnp.uint32(r)) | (x1 >> jnp.uint32(32 - r))
        x1 = x0 ^ x1
    return x0, x1


def _threefry_bits(k1, k2, idx):
    ks2 = k1 ^ k2 ^ jnp.uint32(0x1BD11BDA)
    x0 = jnp.zeros_like(idx) + k1
    x1 = idx + k2
    x0, x1 = _tf_rounds(x0, x1, _ROT_A)
    x0 = x0 + k2
    x1 = x1 + ks2 + jnp.uint32(1)
    x0, x1 = _tf_rounds(x0, x1, _ROT_B)
    x0 = x0 + ks2
    x1 = x1 + k1 + jnp.uint32(2)
    x0, x1 = _tf_rounds(x0, x1, _ROT_A)
    x0 = x0 + k1
    x1 = x1 + k2 + jnp.uint32(3)
    x0, x1 = _tf_rounds(x0, x1, _ROT_B)
    x0 = x0 + k2
    x1 = x1 + ks2 + jnp.uint32(4)
    x0, x1 = _tf_rounds(x0, x1, _ROT_A)
    x0 = x0 + ks2
    x1 = x1 + k1 + jnp.uint32(5)
    return x0 ^ x1


def _sampler_body(vocab, block, nblk, kd_ref, t_ref, logits_ref, tok_ref,
                  lp_ref, gidx, ymax, yidx, ysel, lm, ls):
    j = pl.program_id(0)
    rows = logits_ref.shape[0]

    x = logits_ref[...]
    t = t_ref[...]
    safe_t = jnp.where(t == 0.0, _ONE, t)
    scaled = x / safe_t

    col = jax.lax.broadcasted_iota(jnp.int32, (rows, block), 1) + j * block
    row = jax.lax.broadcasted_iota(jnp.int32, (rows, block), 0)
    valid = col < vocab

    k1 = kd_ref[0]
    k2 = kd_ref[1]
    idx = (row * vocab + col).astype(jnp.uint32)
    bits = _threefry_bits(k1, k2, idx)
    fbits = (bits >> jnp.uint32(9)) | jnp.uint32(0x3F800000)
    u01 = jax.lax.bitcast_convert_type(fbits, jnp.float32) - _ONE
    u = jnp.maximum(_TINY, u01 * (_ONE - _TINY) + _TINY)
    g = -jnp.log(-jnp.log(u))

    sm = jnp.where(valid, scaled, _NEG_INF)
    ym = sm + g

    big = jnp.int32(0x7FFFFFFF)
    bm = jnp.max(sm, axis=1, keepdims=True)
    bxi = jnp.min(jnp.where(sm == bm, col, big), axis=1, keepdims=True)
    by = jnp.max(ym, axis=1, keepdims=True)
    byi = jnp.min(jnp.where(ym == by, col, big), axis=1, keepdims=True)
    bysel = jnp.max(jnp.where(col == byi, sm, _NEG_INF), axis=1, keepdims=True)
    bs = jnp.sum(jnp.exp(sm - bm), axis=1, keepdims=True)

    @pl.when(j == 0)
    def _init():
        gidx[...] = bxi
        ymax[...] = by
        yidx[...] = byi
        ysel[...] = bysel
        lm[...] = bm
        ls[...] = bs

    @pl.when(j > 0)
    def _merge():
        m_old = lm[...]
        upg = bm > m_old
        gidx[...] = jnp.where(upg, bxi, gidx[...])
        upy = by > ymax[...]
        yidx[...] = jnp.where(upy, byi, yidx[...])
        ysel[...] = jnp.where(upy, bysel, ysel[...])
        ymax[...] = jnp.where(upy, by, ymax[...])
        m_new = jnp.where(upg, bm, m_old)
        ls[...] = ls[...] * jnp.exp(m_old - m_new) + bs * jnp.exp(bm - m_new)
        lm[...] = m_new

    @pl.when(j == nblk - 1)
    def _finish():
        zero_t = t == 0.0
        tok_ref[...] = jnp.where(zero_t, gidx[...], yidx[...])
        sel = jnp.where(zero_t, lm[...], ysel[...])
        log_z = lm[...] + jnp.log(ls[...])
        lp_ref[...] = sel - log_z


def kernel(logits, temperatures, key):
    rows, vocab = logits.shape
    logits = logits.astype(jnp.float32)
    kd = jax.random.key_data(key).astype(jnp.uint32).reshape(2)
    t2 = temperatures.astype(jnp.float32).reshape(rows, 1)

    block = 2048 if vocab > 2048 else max(128, -(-vocab // 128) * 128)
    nblk = -(-vocab // block)

    fn = lambda *a: _sampler_body(vocab, block, nblk, *a)
    tok, lp = pl.pallas_call(
        fn,
        grid=(nblk,),
        in_specs=[
            pl.BlockSpec(memory_space=pltpu.SMEM),
            pl.BlockSpec((rows, 1), lambda j: (0, 0)),
            pl.BlockSpec((rows, block), lambda j: (0, j)),
        ],
        out_specs=[
            pl.BlockSpec((rows, 1), lambda j: (0, 0)),
            pl.BlockSpec((rows, 1), lambda j: (0, 0)),
        ],
        out_shape=[
            jax.ShapeDtypeStruct((rows, 1), jnp.int32),
            jax.ShapeDtypeStruct((rows, 1), jnp.float32),
        ],
        scratch_shapes=[
            pltpu.VMEM((rows, 1), jnp.int32),
            pltpu.VMEM((rows, 1), jnp.float32),
            pltpu.VMEM((rows, 1), jnp.int32),
            pltpu.VMEM((rows, 1), jnp.float32),
            pltpu.VMEM((rows, 1), jnp.float32),
            pltpu.VMEM((rows, 1), jnp.float32),
        ],
        compiler_params=pltpu.CompilerParams(
            dimension_semantics=("arbitrary",),
        ),
    )(kd, t2, logits)
    return tok.reshape(rows), lp.reshape(rows)
